# baseline (speedup 1.0000x reference)
import functools

import jax
import jax.numpy as jnp
from jax import lax
from jax.experimental import pallas as pl
from jax.experimental.pallas import tpu as pltpu
from jax.experimental.pallas import tpu_sc as plsc

_info = plsc.get_sparse_core_info()
_NC, _NS = _info.num_cores, _info.num_subcores
_NW = _NC * _NS

_CHUNK_S = 8
_NBUF = 2


@functools.partial(jax.jit, static_argnums=(2, 3, 4))
def _sc_embed(idx, table, b, s, d):
    bpw = (b * s) // _NW
    spw = b // _NW
    chunk = _CHUNK_S * s
    n_chunks = bpw // chunk
    n_outer = n_chunks // _NBUF
    assert bpw % chunk == 0 and n_chunks % _NBUF == 0
    mesh = plsc.VectorSubcoreMesh(core_axis_name="c", subcore_axis_name="s")

    @functools.partial(
        pl.kernel,
        mesh=mesh,
        out_type=jax.ShapeDtypeStruct((b, s, d), jnp.float32),
        scratch_types=[
            pltpu.VMEM((bpw,), jnp.int32),
            pltpu.VMEM((_NBUF, chunk, d), jnp.float32),
            pltpu.SemaphoreType.DMA,
            pltpu.SemaphoreType.DMA,
            pltpu.SemaphoreType.DMA,
        ],
        compiler_params=pltpu.CompilerParams(use_tc_tiling_on_sc=True),
    )
    def k(idx_hbm, table_hbm, out_hbm, idx_v, rows_v, s0, s1, ws):
        gsems = (s0, s1)
        wid = lax.axis_index("s") * _NC + lax.axis_index("c")
        base = wid * bpw
        sbase = wid * spw
        pltpu.sync_copy(idx_hbm.at[pl.ds(base, bpw)], idx_v)

        def body(i, carry):
            toff = i * (_NBUF * chunk)
            gets = []
            for bb in range(_NBUF):
                off = pl.multiple_of(toff + bb * chunk, 8)
                gets.append(
                    pltpu.async_copy(
                        table_hbm.at[idx_v.at[pl.ds(off, chunk)]],
                        rows_v.at[bb],
                        gsems[bb],
                    )
                )
            puts = []
            for bb in range(_NBUF):
                sent0 = sbase + i * (_NBUF * _CHUNK_S) + bb * _CHUNK_S
                gets[bb].wait()
                for j in range(_CHUNK_S):
                    puts.append(
                        pltpu.async_copy(
                            rows_v.at[bb, pl.ds(j * s, s)],
                            out_hbm.at[sent0 + j],
                            ws,
                        )
                    )
            for p in puts:
                p.wait()
            return carry

        lax.fori_loop(0, n_outer, body, 0)

    return k(idx, table)


def kernel(token_ids, embed_matrix):
    b, s = token_ids.shape
    v, d = embed_matrix.shape
    flat = token_ids.reshape(-1).astype(jnp.int32)
    return _sc_embed(flat, embed_matrix, b, s, d)


# speedup vs baseline: 1.7420x; 1.7420x over previous
import functools

import jax
import jax.numpy as jnp
from jax import lax
from jax.experimental import pallas as pl
from jax.experimental.pallas import tpu as pltpu
from jax.experimental.pallas import tpu_sc as plsc

_info = plsc.get_sparse_core_info()
_NC, _NS = _info.num_cores, _info.num_subcores
_NW = _NC * _NS

_CHUNK = 400
_NBUF = 2


@functools.partial(jax.jit, static_argnums=(2, 3))
def _sc_gather(idx, table, bpw, d):
    n_chunks = bpw // _CHUNK
    n_outer = n_chunks // _NBUF
    assert bpw % _CHUNK == 0 and n_chunks % _NBUF == 0
    mesh = plsc.VectorSubcoreMesh(core_axis_name="c", subcore_axis_name="s")

    @functools.partial(
        pl.kernel,
        mesh=mesh,
        out_type=jax.ShapeDtypeStruct((idx.shape[0], d), jnp.float32),
        scratch_types=[
            pltpu.VMEM((bpw,), jnp.int32),
            pltpu.VMEM((_NBUF, _CHUNK, d), jnp.float32),
            pltpu.SemaphoreType.DMA,
            pltpu.SemaphoreType.DMA,
            pltpu.SemaphoreType.DMA,
        ],
    )
    def k(idx_hbm, table_hbm, out_hbm, idx_v, rows_v, s0, s1, ws):
        gsems = (s0, s1)
        wid = lax.axis_index("s") * _NC + lax.axis_index("c")
        base = wid * bpw
        pltpu.sync_copy(idx_hbm.at[pl.ds(base, bpw)], idx_v)

        def body(i, carry):
            ioff = i * (_NBUF * _CHUNK)
            gets = []
            for b in range(_NBUF):
                off = pl.multiple_of(ioff + b * _CHUNK, 8)
                gets.append(
                    pltpu.async_copy(
                        table_hbm.at[idx_v.at[pl.ds(off, _CHUNK)]],
                        rows_v.at[b],
                        gsems[b],
                    )
                )
            puts = []
            for b in range(_NBUF):
                off = pl.multiple_of(ioff + b * _CHUNK, 8)
                gets[b].wait()
                puts.append(
                    pltpu.async_copy(
                        rows_v.at[b], out_hbm.at[pl.ds(base + off, _CHUNK)], ws
                    )
                )
            for p in puts:
                p.wait()
            return carry

        lax.fori_loop(0, n_outer, body, 0)

    return k(idx, table)


def kernel(token_ids, embed_matrix):
    b, s = token_ids.shape
    v, d = embed_matrix.shape
    flat = token_ids.T.reshape(-1).astype(jnp.int32)
    bpw = flat.shape[0] // _NW
    out = _sc_gather(flat, embed_matrix, bpw, d)
    return out.reshape(s, b, d).transpose(1, 0, 2)
